# Initial kernel scaffold; baseline (speedup 1.0000x reference)
import jax
import jax.numpy as jnp
from jax.experimental import pallas as pl


def kernel(facts, beta, aggregator_logits, rule_strength_raw, W, gamma, ln_beta):
    raise NotImplementedError("write your pallas kernel here")


# speedup vs baseline: 12.6180x; 12.6180x over previous
import functools

import jax
import jax.numpy as jnp
from jax.experimental import pallas as pl

_TOP_K = 8
_NEG = -1e30


def _body(facts_ref, beta_ref, alT_ref, rs_ref, W_ref, gamma_ref, lnb_ref,
          out_ref):
    facts = facts_ref[...]
    mask = (beta_ref[...] > 0.0).astype(jnp.float32)

    dn = (((1,), (1,)), ((), ()))
    mm = functools.partial(jax.lax.dot_general, dimension_numbers=dn,
                           preferred_element_type=jnp.float32,
                           precision=jax.lax.Precision.HIGHEST)

    log_f = jnp.log(jnp.maximum(facts, 1e-30))
    log_1mf = jnp.log(jnp.maximum(1.0 - facts, 1e-30))
    s_sum = mm(facts, mask)
    and_agg = jnp.exp(mm(log_f, mask))
    or_agg = 1.0 - jnp.exp(mm(log_1mf, mask))
    cnt = jnp.sum(mask, axis=1)[None, :] + 1e-08
    k_of_n = s_sum / cnt

    alT = alT_ref[...]
    w = jax.nn.softmax(alT, axis=0)
    mixed = (w[0][None, :] * and_agg + w[1][None, :] * or_agg
             + w[2][None, :] * k_of_n + w[3][None, :] * (1.0 - k_of_n))
    act = mixed * jax.nn.sigmoid(rs_ref[...])

    iota = jax.lax.broadcasted_iota(jnp.int32, act.shape, 1)
    a = act
    gate = jnp.zeros_like(act)
    for _ in range(_TOP_K):
        m = jnp.max(a, axis=1, keepdims=True)
        idx = jnp.min(jnp.where(a == m, iota, act.shape[1]), axis=1,
                      keepdims=True)
        sel = iota == idx
        gate = jnp.where(sel, 1.0, gate)
        a = jnp.where(sel, _NEG, a)

    pre = mm(facts, W_ref[...]) + act * gate
    mu = jnp.mean(pre, axis=1, keepdims=True)
    var = jnp.mean((pre - mu) ** 2, axis=1, keepdims=True)
    out_ref[...] = ((pre - mu) * jax.lax.rsqrt(var + 1e-05)
                    * gamma_ref[...] + lnb_ref[...])


def kernel(facts, beta, aggregator_logits, rule_strength_raw, W, gamma,
           ln_beta):
    B, _ = facts.shape
    R, _ = beta.shape
    return pl.pallas_call(
        _body,
        out_shape=jax.ShapeDtypeStruct((B, R), jnp.float32),
    )(facts, beta, aggregator_logits.T, rule_strength_raw[None, :], W,
      gamma[None, :], ln_beta[None, :])
